# baseline (speedup 1.0000x reference)
import functools

import jax
import jax.numpy as jnp
from jax import lax
from jax.experimental import pallas as pl
from jax.experimental.pallas import tpu as pltpu
from jax.experimental.pallas import tpu_sc as plsc

N = 10000
E = 320000
NFEAT = 128
NCLASS = 32
K = 16

NC = 2
NS = 16
L = 16
FH = NCLASS // NC
EPT = E // NS
GSZ = 512
NGRP = 40
EPAD = NGRP * GSZ
RPT = N // NS
MROWS = 1000


def _matmul_body(x_ref, w_ref, o_ref):
    o_ref[...] = lax.dot_general(
        x_ref[...], w_ref[...], (((1,), (1,)), ((), ())),
        preferred_element_type=jnp.float32)


def _project(x, w):
    return pl.pallas_call(
        _matmul_body,
        grid=(N // MROWS,),
        in_specs=[
            pl.BlockSpec((MROWS, NFEAT), lambda i: (i, 0)),
            pl.BlockSpec((NCLASS, NFEAT), lambda i: (0, 0)),
        ],
        out_specs=pl.BlockSpec((MROWS, NCLASS), lambda i: (i, 0)),
        out_shape=jax.ShapeDtypeStruct((N, NCLASS), jnp.float32),
    )(x, w)


def _sc_body(y0_hbm, src_hbm, dst_hbm, w_hbm, bias_hbm, out_hbm,
             src_v, dst_v, w_v, rows3, tmp_v, bias_v, ya, yb,
             gsem0, gsem1, gsem2, gsem3, ssem0, ssem1, ssem2, ssem3):
    c = lax.axis_index("c")
    s = lax.axis_index("s")
    r0 = s * RPT
    gsems = (gsem0, gsem1, gsem2, gsem3)
    ssems = (ssem0, ssem1, ssem2, ssem3)

    pltpu.sync_copy(src_hbm.at[s], src_v)
    pltpu.sync_copy(dst_hbm.at[s], dst_v)
    pltpu.sync_copy(w_hbm.at[s], w_v)
    pltpu.sync_copy(bias_hbm.at[c], bias_v)

    pltpu.sync_copy(y0_hbm.at[pl.ds(r0, RPT), pl.ds(c * FH, FH)], tmp_v)
    pltpu.sync_copy(tmp_v, ya.at[pl.ds(r0, RPT)])
    plsc.subcore_barrier()

    def _fire_g(cur, g, u):
        pltpu.async_copy(cur.at[src_v.at[g]], rows3.at[u], gsems[u])

    def _drain_g(cur, g, u):
        pltpu.make_async_copy(cur.at[src_v.at[g]], rows3.at[u],
                              gsems[u]).wait()

    def _fire_s(nxt, g, u):
        pltpu.async_copy(rows3.at[u], nxt.at[dst_v.at[g]], ssems[u],
                         add=True)

    def _drain_s(nxt, g, u):
        pltpu.make_async_copy(rows3.at[u], nxt.at[dst_v.at[g]],
                              ssems[u]).wait()

    def _mult(g, u):
        rows_u = rows3.at[u]

        def _mul16(g2, carry):
            wvec = w_v[g, pl.ds(g2 * L, L)]
            for t in range(L):
                rows_u[g2 * L + t] = rows_u[g2 * L + t] * wvec[t]
            return carry
        lax.fori_loop(0, GSZ // L, _mul16, 0)

    def _step(cur, nxt):
        pltpu.sync_copy(cur.at[pl.ds(r0, RPT)], nxt.at[pl.ds(r0, RPT)])
        plsc.subcore_barrier()

        _fire_g(cur, 0, 0)
        _fire_g(cur, 1, 1)
        _fire_g(cur, 2, 2)
        _drain_g(cur, 0, 0)
        _mult(0, 0)
        _fire_s(nxt, 0, 0)
        _fire_g(cur, 3, 3)
        _drain_g(cur, 1, 1)
        _mult(1, 1)
        _fire_s(nxt, 1, 1)

        def _quad(t, carry):
            for uoff in range(4):
                g = 2 + t * 4 + uoff
                u = (2 + uoff) % 4
                _drain_s(nxt, g - 2, (u + 2) % 4)
                _fire_g(cur, g + 2, (u + 2) % 4)
                _drain_g(cur, g, u)
                _mult(g, u)
                _fire_s(nxt, g, u)
            return carry
        lax.fori_loop(0, (NGRP - 4) // 4, _quad, 0)

        _drain_s(nxt, 36, 0)
        _drain_g(cur, 38, 2)
        _mult(38, 2)
        _fire_s(nxt, 38, 2)
        _drain_s(nxt, 37, 1)
        _drain_g(cur, 39, 3)
        _mult(39, 3)
        _fire_s(nxt, 39, 3)
        _drain_s(nxt, 38, 2)
        _drain_s(nxt, 39, 3)
        plsc.subcore_barrier()

    def _two_steps(k, carry):
        _step(ya, yb)
        _step(yb, ya)
        return carry
    lax.fori_loop(0, K // 2, _two_steps, 0)

    pltpu.sync_copy(ya.at[pl.ds(r0, RPT)], tmp_v)
    bvec = bias_v[...]

    def _out(i, carry):
        tmp_v[i] = tmp_v[i] + bvec
        return carry
    lax.fori_loop(0, RPT, _out, 0)
    pltpu.sync_copy(tmp_v, out_hbm.at[pl.ds(r0, RPT), pl.ds(c * FH, FH)])


_sc_propagate = pl.kernel(
    _sc_body,
    out_type=jax.ShapeDtypeStruct((N, NCLASS), jnp.float32),
    mesh=plsc.VectorSubcoreMesh(core_axis_name="c", subcore_axis_name="s"),
    compiler_params=pltpu.CompilerParams(use_tc_tiling_on_sc=False),
    scratch_types=[
        pltpu.VMEM((NGRP, GSZ), jnp.int32),
        pltpu.VMEM((NGRP, GSZ), jnp.int32),
        pltpu.VMEM((NGRP, GSZ), jnp.float32),
        pltpu.VMEM((4, GSZ, L), jnp.float32),
        pltpu.VMEM((RPT, FH), jnp.float32),
        pltpu.VMEM((FH,), jnp.float32),
        pltpu.VMEM_SHARED((N, FH), jnp.float32),
        pltpu.VMEM_SHARED((N, FH), jnp.float32),
        pltpu.SemaphoreType.DMA,
        pltpu.SemaphoreType.DMA,
        pltpu.SemaphoreType.DMA,
        pltpu.SemaphoreType.DMA,
        pltpu.SemaphoreType.DMA,
        pltpu.SemaphoreType.DMA,
        pltpu.SemaphoreType.DMA,
        pltpu.SemaphoreType.DMA,
    ],
)


def kernel(x, edge_index, edge_weight, T, W_weight, W_bias):
    delta = (T / K).astype(jnp.float32)
    omd = (1.0 - delta).astype(jnp.float32)

    y0 = _project(x, W_weight * (omd ** K))

    src = edge_index[0].reshape(NS, EPT)
    dst = edge_index[1].reshape(NS, EPT)
    wsc = (edge_weight * (delta / omd)).reshape(NS, EPT)
    pad = ((0, 0), (0, EPAD - EPT))
    src3 = jnp.pad(src, pad).reshape(NS, NGRP, GSZ)
    dst3 = jnp.pad(dst, pad).reshape(NS, NGRP, GSZ)
    w3 = jnp.pad(wsc, pad).reshape(NS, NGRP, GSZ)

    bias2 = W_bias.reshape(NC, FH)

    return _sc_propagate(y0, src3, dst3, w3, bias2)


# speedup vs baseline: 1.4138x; 1.4138x over previous
import functools

import jax
import jax.numpy as jnp
from jax import lax
from jax.experimental import pallas as pl
from jax.experimental.pallas import tpu as pltpu
from jax.experimental.pallas import tpu_sc as plsc

N = 10000
E = 320000
NFEAT = 128
NCLASS = 32
K = 16

NC = 2
NS = 16
L = 16
FH = NCLASS // NC
EPT = E // NS
GSZ = 512
NGRP = 40
EPAD = NGRP * GSZ
RPT = N // NS
MROWS = 1000


def _matmul_body(x_ref, w_ref, o_ref):
    o_ref[...] = lax.dot_general(
        x_ref[...], w_ref[...], (((1,), (1,)), ((), ())),
        preferred_element_type=jnp.float32)


def _project(x, w):
    return pl.pallas_call(
        _matmul_body,
        grid=(N // MROWS,),
        in_specs=[
            pl.BlockSpec((MROWS, NFEAT), lambda i: (i, 0)),
            pl.BlockSpec((NCLASS, NFEAT), lambda i: (0, 0)),
        ],
        out_specs=pl.BlockSpec((MROWS, NCLASS), lambda i: (i, 0)),
        out_shape=jax.ShapeDtypeStruct((N, NCLASS), jnp.float32),
    )(x, w)


def _sc_body(y0_hbm, src_hbm, dst_hbm, w_hbm, bias_hbm, out_hbm,
             src_v, dst_v, w_v, rows3, tmp_v, bias_v, ya, yb,
             gsem0, gsem1, gsem2, gsem3, ssem0, ssem1, ssem2, ssem3):
    c = lax.axis_index("c")
    s = lax.axis_index("s")
    r0 = s * RPT
    gsems = (gsem0, gsem1, gsem2, gsem3)
    ssems = (ssem0, ssem1, ssem2, ssem3)

    pltpu.sync_copy(src_hbm.at[s], src_v)
    pltpu.sync_copy(dst_hbm.at[s], dst_v)
    pltpu.sync_copy(w_hbm.at[s], w_v)
    pltpu.sync_copy(bias_hbm.at[c], bias_v)

    pltpu.sync_copy(y0_hbm.at[pl.ds(r0, RPT), pl.ds(c * FH, FH)], tmp_v)
    pltpu.sync_copy(tmp_v, ya.at[pl.ds(r0, RPT)])
    plsc.subcore_barrier()

    def _fire_g(cur, g, u):
        pltpu.async_copy(cur.at[src_v.at[g]], rows3.at[u], gsems[u])

    def _drain_g(cur, g, u):
        pltpu.make_async_copy(cur.at[src_v.at[g]], rows3.at[u],
                              gsems[u]).wait()

    def _fire_s(nxt, g, u):
        pltpu.async_copy(rows3.at[u], nxt.at[dst_v.at[g]], ssems[u],
                         add=True)

    def _drain_s(nxt, g, u):
        pltpu.make_async_copy(rows3.at[u], nxt.at[dst_v.at[g]],
                              ssems[u]).wait()

    def _mult(g, u):
        rows_u = rows3.at[u]

        def _mul16(g2, carry):
            wvec = w_v[g, pl.ds(g2 * L, L)]
            for t in range(L):
                rows_u[g2 * L + t] = rows_u[g2 * L + t] * wvec[t]
            return carry
        lax.fori_loop(0, GSZ // L, _mul16, 0)

    def _step(cur, nxt):
        pltpu.sync_copy(cur.at[pl.ds(r0, RPT)], tmp_v)
        pltpu.sync_copy(tmp_v, nxt.at[pl.ds(r0, RPT)])
        plsc.subcore_barrier()

        _fire_g(cur, 0, 0)
        _fire_g(cur, 1, 1)
        _fire_g(cur, 2, 2)
        _drain_g(cur, 0, 0)
        _mult(0, 0)
        _fire_s(nxt, 0, 0)
        _fire_g(cur, 3, 3)
        _drain_g(cur, 1, 1)
        _mult(1, 1)
        _fire_s(nxt, 1, 1)

        def _quad(t, carry):
            for uoff in range(4):
                g = 2 + t * 4 + uoff
                u = (2 + uoff) % 4
                _drain_s(nxt, g - 2, (u + 2) % 4)
                _fire_g(cur, g + 2, (u + 2) % 4)
                _drain_g(cur, g, u)
                _mult(g, u)
                _fire_s(nxt, g, u)
            return carry
        lax.fori_loop(0, (NGRP - 4) // 4, _quad, 0)

        _drain_s(nxt, 36, 0)
        _drain_g(cur, 38, 2)
        _mult(38, 2)
        _fire_s(nxt, 38, 2)
        _drain_s(nxt, 37, 1)
        _drain_g(cur, 39, 3)
        _mult(39, 3)
        _fire_s(nxt, 39, 3)
        _drain_s(nxt, 38, 2)
        _drain_s(nxt, 39, 3)
        plsc.subcore_barrier()

    def _two_steps(k, carry):
        _step(ya, yb)
        _step(yb, ya)
        return carry
    lax.fori_loop(0, K // 2, _two_steps, 0)

    pltpu.sync_copy(ya.at[pl.ds(r0, RPT)], tmp_v)
    bvec = bias_v[...]

    def _out(i, carry):
        tmp_v[i] = tmp_v[i] + bvec
        return carry
    lax.fori_loop(0, RPT, _out, 0)
    pltpu.sync_copy(tmp_v, out_hbm.at[pl.ds(r0, RPT), pl.ds(c * FH, FH)])


_sc_propagate = pl.kernel(
    _sc_body,
    out_type=jax.ShapeDtypeStruct((N, NCLASS), jnp.float32),
    mesh=plsc.VectorSubcoreMesh(core_axis_name="c", subcore_axis_name="s"),
    compiler_params=pltpu.CompilerParams(use_tc_tiling_on_sc=False),
    scratch_types=[
        pltpu.VMEM((NGRP, GSZ), jnp.int32),
        pltpu.VMEM((NGRP, GSZ), jnp.int32),
        pltpu.VMEM((NGRP, GSZ), jnp.float32),
        pltpu.VMEM((4, GSZ, L), jnp.float32),
        pltpu.VMEM((RPT, FH), jnp.float32),
        pltpu.VMEM((FH,), jnp.float32),
        pltpu.VMEM_SHARED((N, FH), jnp.float32),
        pltpu.VMEM_SHARED((N, FH), jnp.float32),
        pltpu.SemaphoreType.DMA,
        pltpu.SemaphoreType.DMA,
        pltpu.SemaphoreType.DMA,
        pltpu.SemaphoreType.DMA,
        pltpu.SemaphoreType.DMA,
        pltpu.SemaphoreType.DMA,
        pltpu.SemaphoreType.DMA,
        pltpu.SemaphoreType.DMA,
    ],
)


def kernel(x, edge_index, edge_weight, T, W_weight, W_bias):
    delta = (T / K).astype(jnp.float32)
    omd = (1.0 - delta).astype(jnp.float32)

    y0 = _project(x, W_weight * (omd ** K))

    src = edge_index[0].reshape(NS, EPT)
    dst = edge_index[1].reshape(NS, EPT)
    wsc = (edge_weight * (delta / omd)).reshape(NS, EPT)
    pad = ((0, 0), (0, EPAD - EPT))
    src3 = jnp.pad(src, pad).reshape(NS, NGRP, GSZ)
    dst3 = jnp.pad(dst, pad).reshape(NS, NGRP, GSZ)
    w3 = jnp.pad(wsc, pad).reshape(NS, NGRP, GSZ)

    bias2 = W_bias.reshape(NC, FH)

    return _sc_propagate(y0, src3, dst3, w3, bias2)
